# Initial kernel scaffold; baseline (speedup 1.0000x reference)
import jax
import jax.numpy as jnp
from jax.experimental import pallas as pl


def kernel(node_feature, adj_list, W1, b1, Wl1, bl1, W2, b2, Wl2, bl2, W3, b3, Wl3, bl3):
    raise NotImplementedError("write your pallas kernel here")


# speedup vs baseline: 11.6620x; 11.6620x over previous
import functools

import jax
import jax.numpy as jnp
from jax import lax
from jax.experimental import pallas as pl
from jax.experimental.pallas import tpu as pltpu
from jax.experimental.pallas import tpu_sc as plsc

N = 10000
E = 320000
D_IN = 128
D_MID = 128
D_MID2 = 64
D_OUT = 128

NC = 2
NS = 16
TILES = NC * NS
E_PER_TILE = E // TILES
RS = 624
TAIL = N - NS * RS
CH = 80
N_CHUNKS = E_PER_TILE // CH
ZR = 208

_mesh = lambda: plsc.VectorSubcoreMesh(core_axis_name="c", subcore_axis_name="s")


def _sc_degree(dst):
    @functools.partial(
        pl.kernel,
        mesh=_mesh(),
        out_type=jax.ShapeDtypeStruct((NC, N), jnp.float32),
        scratch_types=[
            pltpu.VMEM((CH,), jnp.int32),
            pltpu.VMEM((CH,), jnp.float32),
            pltpu.VMEM((1008,), jnp.float32),
            pltpu.VMEM_SHARED((N,), jnp.float32),
        ],
    )
    def k(dst_hbm, out_hbm, dstv, ones, zbuf, acc):
        c = lax.axis_index("c")
        s = lax.axis_index("s")
        wid = c * NS + s

        def zfill(j, carry):
            zbuf[pl.ds(j * 16, 16)] = jnp.zeros((16,), jnp.float32)
            return carry

        lax.fori_loop(0, 63, zfill, 0)

        def ofill(j, carry):
            ones[pl.ds(j * 16, 16)] = jnp.ones((16,), jnp.float32)
            return carry

        lax.fori_loop(0, CH // 16, ofill, 0)

        @pl.when(s < 10)
        def _():
            pltpu.sync_copy(zbuf.at[pl.ds(0, 1000)], acc.at[pl.ds(s * 1000, 1000)])

        plsc.subcore_barrier()

        ebase = wid * E_PER_TILE

        def body(i, carry):
            pltpu.sync_copy(dst_hbm.at[pl.ds(ebase + i * CH, CH)], dstv)
            pltpu.sync_copy(ones, acc.at[dstv], add=True)
            return carry

        lax.fori_loop(0, N_CHUNKS, body, 0)
        plsc.subcore_barrier()

        @pl.when(s == 0)
        def _():
            pltpu.sync_copy(acc, out_hbm.at[c])

    return k(dst)


def _make_sc_agg(d):
    @functools.partial(
        pl.kernel,
        mesh=_mesh(),
        out_type=jax.ShapeDtypeStruct((NC, N, d), jnp.float32),
        scratch_types=[
            pltpu.VMEM((CH,), jnp.int32),
            pltpu.VMEM((CH,), jnp.int32),
            pltpu.VMEM((CH, d), jnp.float32),
            pltpu.VMEM((ZR, d), jnp.float32),
            pltpu.VMEM_SHARED((N, d), jnp.float32),
            pltpu.SemaphoreType.DMA,
        ],
    )
    def k(y_hbm, src_hbm, dst_hbm, out_hbm, srcv, dstv, rows, zbuf, acc, sem):
        c = lax.axis_index("c")
        s = lax.axis_index("s")
        wid = c * NS + s

        def zrow(r, carry):
            def zcol(j, carry2):
                zbuf[r, pl.ds(j * 16, 16)] = jnp.zeros((16,), jnp.float32)
                return carry2

            return lax.fori_loop(0, d // 16, zcol, carry)

        lax.fori_loop(0, ZR, zrow, 0)

        def zcopy(t, carry):
            pltpu.sync_copy(zbuf, acc.at[pl.ds(s * RS + t * ZR, ZR)])
            return carry

        lax.fori_loop(0, RS // ZR, zcopy, 0)

        @pl.when(s == 0)
        def _():
            pltpu.sync_copy(zbuf.at[pl.ds(0, TAIL)], acc.at[pl.ds(NS * RS, TAIL)])

        plsc.subcore_barrier()

        ebase = wid * E_PER_TILE

        def body(i, carry):
            off = ebase + i * CH
            pltpu.sync_copy(src_hbm.at[pl.ds(off, CH)], srcv)
            pltpu.sync_copy(dst_hbm.at[pl.ds(off, CH)], dstv)
            pltpu.async_copy(y_hbm.at[srcv], rows, sem).wait()
            pltpu.sync_copy(rows, acc.at[dstv], add=True)
            return carry

        lax.fori_loop(0, N_CHUNKS, body, 0)
        plsc.subcore_barrier()

        pltpu.sync_copy(
            acc.at[pl.ds(s * RS, RS)],
            out_hbm.at[c, pl.ds(s * RS, RS), :],
        )

        @pl.when(s == 0)
        def _():
            pltpu.sync_copy(
                acc.at[pl.ds(NS * RS, TAIL)],
                out_hbm.at[c, pl.ds(NS * RS, TAIL), :],
            )

    return k


_sc_agg128 = _make_sc_agg(D_MID)


BN = 1000
GRID = N // BN


def _elu(a):
    return jnp.where(a > 0.0, a, jnp.exp(jnp.minimum(a, 0.0)) - 1.0)


def _rows(i):
    return (i, 0)


def _fixed(i):
    return (0, 0)


def _rows3(i):
    return (0, i, 0)


def _tc1(deg, x, W1, Wl1, bl1):
    def body(deg_r, x_r, w_r, wl_r, bl_r, y_r, skip_r, dinv_r):
        dinv = lax.rsqrt(deg_r[...])
        xb = x_r[...]
        y_r[...] = dinv * jnp.dot(xb, w_r[...], preferred_element_type=jnp.float32)
        skip_r[...] = jnp.dot(xb, wl_r[...], preferred_element_type=jnp.float32) + bl_r[...]
        dinv_r[...] = dinv

    return pl.pallas_call(
        body,
        grid=(GRID,),
        in_specs=[
            pl.BlockSpec((BN, 1), _rows),
            pl.BlockSpec((BN, D_IN), _rows),
            pl.BlockSpec((D_IN, D_MID), _fixed),
            pl.BlockSpec((D_IN, D_MID), _fixed),
            pl.BlockSpec((1, D_MID), _fixed),
        ],
        out_specs=[
            pl.BlockSpec((BN, D_MID), _rows),
            pl.BlockSpec((BN, D_MID), _rows),
            pl.BlockSpec((BN, 1), _rows),
        ],
        out_shape=[
            jax.ShapeDtypeStruct((N, D_MID), jnp.float32),
            jax.ShapeDtypeStruct((N, D_MID), jnp.float32),
            jax.ShapeDtypeStruct((N, 1), jnp.float32),
        ],
    )(deg, x, W1, Wl1, bl1)


def _make_tc_mid(d_in, d_out):
    def body(u_r, y_r, dinv_r, skip_r, b_r, w_r, wl_r, bl_r, y2_r, skip2_r):
        dinv = dinv_r[...]
        u = u_r[0] + u_r[1]
        h = _elu(dinv * (u + y_r[...]) + b_r[...]) + skip_r[...]
        y2_r[...] = dinv * jnp.dot(h, w_r[...], preferred_element_type=jnp.float32)
        skip2_r[...] = jnp.dot(h, wl_r[...], preferred_element_type=jnp.float32) + bl_r[...]

    def run(u, y, dinv, skip, b, W, Wl, bl):
        return pl.pallas_call(
            body,
            grid=(GRID,),
            in_specs=[
                pl.BlockSpec((NC, BN, d_in), _rows3),
                pl.BlockSpec((BN, d_in), _rows),
                pl.BlockSpec((BN, 1), _rows),
                pl.BlockSpec((BN, d_in), _rows),
                pl.BlockSpec((1, d_in), _fixed),
                pl.BlockSpec((d_in, d_out), _fixed),
                pl.BlockSpec((d_in, d_out), _fixed),
                pl.BlockSpec((1, d_out), _fixed),
            ],
            out_specs=[
                pl.BlockSpec((BN, d_out), _rows),
                pl.BlockSpec((BN, d_out), _rows),
            ],
            out_shape=[
                jax.ShapeDtypeStruct((N, d_out), jnp.float32),
                jax.ShapeDtypeStruct((N, d_out), jnp.float32),
            ],
        )(u, y, dinv, skip, b, W, Wl, bl)

    return run


_tc_mid = _make_tc_mid(D_MID, D_MID)


def _tc_final(u, y, dinv, skip, b):
    def body(u_r, y_r, dinv_r, skip_r, b_r, o_r):
        u_ = u_r[0] + u_r[1]
        o_r[...] = dinv_r[...] * (u_ + y_r[...]) + b_r[...] + skip_r[...]

    return pl.pallas_call(
        body,
        grid=(GRID,),
        in_specs=[
            pl.BlockSpec((NC, BN, D_OUT), _rows3),
            pl.BlockSpec((BN, D_OUT), _rows),
            pl.BlockSpec((BN, 1), _rows),
            pl.BlockSpec((BN, D_OUT), _rows),
            pl.BlockSpec((1, D_OUT), _fixed),
        ],
        out_specs=pl.BlockSpec((BN, D_OUT), _rows),
        out_shape=jax.ShapeDtypeStruct((N, D_OUT), jnp.float32),
    )(u, y, dinv, skip, b)


def kernel(node_feature, adj_list, W1, b1, Wl1, bl1, W2, b2, Wl2, bl2,
           W3, b3, Wl3, bl3):
    x = node_feature
    src = adj_list[0].astype(jnp.int32)
    dst = adj_list[1].astype(jnp.int32)

    degp = _sc_degree(dst)
    deg = (degp[0] + degp[1] + 1.0).reshape(N, 1)

    pad = D_MID - D_MID2
    W2p = jnp.pad(W2, ((0, 0), (0, pad)))
    Wl2p = jnp.pad(Wl2, ((0, 0), (0, pad)))
    W3p = jnp.pad(W3, ((0, pad), (0, 0)))
    Wl3p = jnp.pad(Wl3, ((0, pad), (0, 0)))
    b1r = b1.reshape(1, -1)
    b2r = jnp.pad(b2, (0, pad)).reshape(1, -1)
    b3r = b3.reshape(1, -1)
    bl1r = bl1.reshape(1, -1)
    bl2r = jnp.pad(bl2, (0, pad)).reshape(1, -1)
    bl3r = bl3.reshape(1, -1)

    y1, skip1, dinv = _tc1(deg, x, W1, Wl1, bl1r)
    u1 = _sc_agg128(y1, src, dst)
    y2, skip2 = _tc_mid(u1, y1, dinv, skip1, b1r, W2p, Wl2p, bl2r)
    u2 = _sc_agg128(y2, src, dst)
    y3, skip3 = _tc_mid(u2, y2, dinv, skip2, b2r, W3p, Wl3p, bl3r)
    u3 = _sc_agg128(y3, src, dst)
    out = _tc_final(u3, y3, dinv, skip3, b3r)
    return out


# speedup vs baseline: 24.6971x; 2.1177x over previous
import functools

import jax
import jax.numpy as jnp
from jax import lax
from jax.experimental import pallas as pl
from jax.experimental.pallas import tpu as pltpu
from jax.experimental.pallas import tpu_sc as plsc

N = 10000
E = 320000
D_IN = 128
D_MID = 128
D_MID2 = 64
D_OUT = 128

NC = 2
NS = 16
TILES = NC * NS
E_PER_TILE = E // TILES
RS = 624
TAIL = N - NS * RS
CH = 80
N_CHUNKS = E_PER_TILE // CH
ZR = 16

_mesh = lambda: plsc.VectorSubcoreMesh(core_axis_name="c", subcore_axis_name="s")


def _sc_degree(dst):
    @functools.partial(
        pl.kernel,
        mesh=_mesh(),
        out_type=jax.ShapeDtypeStruct((NC, N), jnp.float32),
        scratch_types=[
            pltpu.VMEM((CH,), jnp.int32),
            pltpu.VMEM((CH,), jnp.float32),
            pltpu.VMEM((1008,), jnp.float32),
            pltpu.VMEM_SHARED((N,), jnp.float32),
        ],
    )
    def k(dst_hbm, out_hbm, dstv, ones, zbuf, acc):
        c = lax.axis_index("c")
        s = lax.axis_index("s")
        wid = c * NS + s

        def zfill(j, carry):
            zbuf[pl.ds(j * 16, 16)] = jnp.zeros((16,), jnp.float32)
            return carry

        lax.fori_loop(0, 63, zfill, 0)

        def ofill(j, carry):
            ones[pl.ds(j * 16, 16)] = jnp.ones((16,), jnp.float32)
            return carry

        lax.fori_loop(0, CH // 16, ofill, 0)

        @pl.when(s < 10)
        def _():
            pltpu.sync_copy(zbuf.at[pl.ds(0, 1000)], acc.at[pl.ds(s * 1000, 1000)])

        plsc.subcore_barrier()

        ebase = wid * E_PER_TILE

        def body(i, carry):
            pltpu.sync_copy(dst_hbm.at[pl.ds(ebase + i * CH, CH)], dstv)
            pltpu.sync_copy(ones, acc.at[dstv], add=True)
            return carry

        lax.fori_loop(0, N_CHUNKS, body, 0)
        plsc.subcore_barrier()

        @pl.when(s == 0)
        def _():
            pltpu.sync_copy(acc, out_hbm.at[c])

    return k(dst)


def _make_sc_agg(d):
    @functools.partial(
        pl.kernel,
        mesh=_mesh(),
        out_type=jax.ShapeDtypeStruct((NC, N, d), jnp.float32),
        scratch_types=[
            pltpu.VMEM((E_PER_TILE,), jnp.int32),
            pltpu.VMEM((E_PER_TILE,), jnp.int32),
            pltpu.VMEM((CH,), jnp.int32),
            pltpu.VMEM((CH,), jnp.int32),
            pltpu.VMEM((CH,), jnp.int32),
            pltpu.VMEM((CH,), jnp.int32),
            pltpu.VMEM((CH, d), jnp.float32),
            pltpu.VMEM((CH, d), jnp.float32),
            pltpu.VMEM((ZR, d), jnp.float32),
            pltpu.VMEM_SHARED((N, d), jnp.float32),
            pltpu.SemaphoreType.DMA,
            pltpu.SemaphoreType.DMA,
        ],
    )
    def k(y_hbm, src_hbm, dst_hbm, out_hbm, srcv, dstv, sca, scb, dca, dcb,
          rows_a, rows_b, zbuf, acc, sem_a, sem_b):
        c = lax.axis_index("c")
        s = lax.axis_index("s")
        wid = c * NS + s

        idx_cp_s = pltpu.async_copy(src_hbm.at[wid], srcv, sem_a)
        idx_cp_d = pltpu.async_copy(dst_hbm.at[wid], dstv, sem_b)

        def zrow(r, carry):
            def zcol(j, carry2):
                zbuf[r, pl.ds(j * 16, 16)] = jnp.zeros((16,), jnp.float32)
                return carry2

            return lax.fori_loop(0, d // 16, zcol, carry)

        lax.fori_loop(0, ZR, zrow, 0)

        def zcopy(t, carry):
            pltpu.sync_copy(zbuf, acc.at[pl.ds(s * RS + t * ZR, ZR)])
            return carry

        lax.fori_loop(0, RS // ZR, zcopy, 0)

        @pl.when(s == 0)
        def _():
            pltpu.sync_copy(zbuf.at[pl.ds(0, TAIL)], acc.at[pl.ds(NS * RS, TAIL)])

        idx_cp_s.wait()
        idx_cp_d.wait()
        plsc.subcore_barrier()

        def cpidx(i, slab, buf):
            def one(j, carry):
                buf[pl.ds(j * 16, 16)] = slab[pl.ds(i * CH + j * 16, 16)]
                return carry

            lax.fori_loop(0, CH // 16, one, 0)

        def gather(i, sbuf, rows, sem):
            cpidx(i, srcv, sbuf)
            return pltpu.async_copy(y_hbm.at[sbuf], rows, sem)

        def gwait(sbuf, rows, sem):
            pltpu.make_async_copy(y_hbm.at[pl.ds(0, CH)], rows, sem).wait()

        def scat(i, dbuf, rows):
            cpidx(i, dstv, dbuf)
            pltpu.sync_copy(rows, acc.at[dbuf], add=True)

        gather(0, sca, rows_a, sem_a)

        def body(j, carry):
            i0 = 2 * j
            gather(i0 + 1, scb, rows_b, sem_b)
            gwait(sca, rows_a, sem_a)
            scat(i0, dca, rows_a)
            gather(i0 + 2, sca, rows_a, sem_a)
            gwait(scb, rows_b, sem_b)
            scat(i0 + 1, dcb, rows_b)
            return carry

        lax.fori_loop(0, (N_CHUNKS - 1) // 2, body, 0)
        gwait(sca, rows_a, sem_a)
        scat(N_CHUNKS - 1, dca, rows_a)
        plsc.subcore_barrier()

        pltpu.sync_copy(
            acc.at[pl.ds(s * RS, RS)],
            out_hbm.at[c, pl.ds(s * RS, RS), :],
        )

        @pl.when(s == 0)
        def _():
            pltpu.sync_copy(
                acc.at[pl.ds(NS * RS, TAIL)],
                out_hbm.at[c, pl.ds(NS * RS, TAIL), :],
            )

    return k


_sc_agg128 = _make_sc_agg(D_MID)


BN = 1000
GRID = N // BN


def _elu(a):
    return jnp.where(a > 0.0, a, jnp.exp(jnp.minimum(a, 0.0)) - 1.0)


def _rows(i):
    return (i, 0)


def _fixed(i):
    return (0, 0)


def _rows3(i):
    return (0, i, 0)


def _tc1(deg, x, W1, Wl1, bl1):
    def body(deg_r, x_r, w_r, wl_r, bl_r, y_r, skip_r, dinv_r):
        dinv = lax.rsqrt(deg_r[...])
        xb = x_r[...]
        y_r[...] = dinv * jnp.dot(xb, w_r[...], preferred_element_type=jnp.float32)
        skip_r[...] = jnp.dot(xb, wl_r[...], preferred_element_type=jnp.float32) + bl_r[...]
        dinv_r[...] = dinv

    return pl.pallas_call(
        body,
        grid=(GRID,),
        in_specs=[
            pl.BlockSpec((BN, 1), _rows),
            pl.BlockSpec((BN, D_IN), _rows),
            pl.BlockSpec((D_IN, D_MID), _fixed),
            pl.BlockSpec((D_IN, D_MID), _fixed),
            pl.BlockSpec((1, D_MID), _fixed),
        ],
        out_specs=[
            pl.BlockSpec((BN, D_MID), _rows),
            pl.BlockSpec((BN, D_MID), _rows),
            pl.BlockSpec((BN, 1), _rows),
        ],
        out_shape=[
            jax.ShapeDtypeStruct((N, D_MID), jnp.float32),
            jax.ShapeDtypeStruct((N, D_MID), jnp.float32),
            jax.ShapeDtypeStruct((N, 1), jnp.float32),
        ],
    )(deg, x, W1, Wl1, bl1)


def _make_tc_mid(d_in, d_out):
    def body(u_r, y_r, dinv_r, skip_r, b_r, w_r, wl_r, bl_r, y2_r, skip2_r):
        dinv = dinv_r[...]
        u = u_r[0] + u_r[1]
        h = _elu(dinv * (u + y_r[...]) + b_r[...]) + skip_r[...]
        y2_r[...] = dinv * jnp.dot(h, w_r[...], preferred_element_type=jnp.float32)
        skip2_r[...] = jnp.dot(h, wl_r[...], preferred_element_type=jnp.float32) + bl_r[...]

    def run(u, y, dinv, skip, b, W, Wl, bl):
        return pl.pallas_call(
            body,
            grid=(GRID,),
            in_specs=[
                pl.BlockSpec((NC, BN, d_in), _rows3),
                pl.BlockSpec((BN, d_in), _rows),
                pl.BlockSpec((BN, 1), _rows),
                pl.BlockSpec((BN, d_in), _rows),
                pl.BlockSpec((1, d_in), _fixed),
                pl.BlockSpec((d_in, d_out), _fixed),
                pl.BlockSpec((d_in, d_out), _fixed),
                pl.BlockSpec((1, d_out), _fixed),
            ],
            out_specs=[
                pl.BlockSpec((BN, d_out), _rows),
                pl.BlockSpec((BN, d_out), _rows),
            ],
            out_shape=[
                jax.ShapeDtypeStruct((N, d_out), jnp.float32),
                jax.ShapeDtypeStruct((N, d_out), jnp.float32),
            ],
        )(u, y, dinv, skip, b, W, Wl, bl)

    return run


_tc_mid = _make_tc_mid(D_MID, D_MID)


def _tc_final(u, y, dinv, skip, b):
    def body(u_r, y_r, dinv_r, skip_r, b_r, o_r):
        u_ = u_r[0] + u_r[1]
        o_r[...] = dinv_r[...] * (u_ + y_r[...]) + b_r[...] + skip_r[...]

    return pl.pallas_call(
        body,
        grid=(GRID,),
        in_specs=[
            pl.BlockSpec((NC, BN, D_OUT), _rows3),
            pl.BlockSpec((BN, D_OUT), _rows),
            pl.BlockSpec((BN, 1), _rows),
            pl.BlockSpec((BN, D_OUT), _rows),
            pl.BlockSpec((1, D_OUT), _fixed),
        ],
        out_specs=pl.BlockSpec((BN, D_OUT), _rows),
        out_shape=jax.ShapeDtypeStruct((N, D_OUT), jnp.float32),
    )(u, y, dinv, skip, b)


def kernel(node_feature, adj_list, W1, b1, Wl1, bl1, W2, b2, Wl2, bl2,
           W3, b3, Wl3, bl3):
    x = node_feature
    src = adj_list[0].astype(jnp.int32)
    dst = adj_list[1].astype(jnp.int32)
    src3 = src.reshape(TILES, E_PER_TILE)
    dst3 = dst.reshape(TILES, E_PER_TILE)

    degp = _sc_degree(dst)
    deg = (degp[0] + degp[1] + 1.0).reshape(N, 1)

    pad = D_MID - D_MID2
    W2p = jnp.pad(W2, ((0, 0), (0, pad)))
    Wl2p = jnp.pad(Wl2, ((0, 0), (0, pad)))
    W3p = jnp.pad(W3, ((0, pad), (0, 0)))
    Wl3p = jnp.pad(Wl3, ((0, pad), (0, 0)))
    b1r = b1.reshape(1, -1)
    b2r = jnp.pad(b2, (0, pad)).reshape(1, -1)
    b3r = b3.reshape(1, -1)
    bl1r = bl1.reshape(1, -1)
    bl2r = jnp.pad(bl2, (0, pad)).reshape(1, -1)
    bl3r = bl3.reshape(1, -1)

    y1, skip1, dinv = _tc1(deg, x, W1, Wl1, bl1r)
    u1 = _sc_agg128(y1, src3, dst3)
    y2, skip2 = _tc_mid(u1, y1, dinv, skip1, b1r, W2p, Wl2p, bl2r)
    u2 = _sc_agg128(y2, src3, dst3)
    y3, skip3 = _tc_mid(u2, y2, dinv, skip2, b2r, W3p, Wl3p, bl3r)
    u3 = _sc_agg128(y3, src3, dst3)
    out = _tc_final(u3, y3, dinv, skip3, b3r)
    return out


# speedup vs baseline: 30.6734x; 1.2420x over previous
import functools

import jax
import jax.numpy as jnp
from jax import lax
from jax.experimental import pallas as pl
from jax.experimental.pallas import tpu as pltpu
from jax.experimental.pallas import tpu_sc as plsc

N = 10000
E = 320000
D_IN = 128
D_MID = 128
D_MID2 = 64
D_OUT = 128

NC = 2
NS = 16
TILES = NC * NS
E_PER_TILE = E // TILES
RS = 624
TAIL = N - NS * RS
CH = 128
N_CHUNKS = E_PER_TILE // CH
ECH_TAIL = E_PER_TILE - N_CHUNKS * CH
ZR = 16
NZ = RS // ZR

_mesh = lambda: plsc.VectorSubcoreMesh(core_axis_name="c", subcore_axis_name="s")


def _sc_degree(dst):
    @functools.partial(
        pl.kernel,
        mesh=_mesh(),
        out_type=jax.ShapeDtypeStruct((NC, N), jnp.float32),
        scratch_types=[
            pltpu.VMEM((E_PER_TILE,), jnp.int32),
            pltpu.VMEM((CH,), jnp.int32),
            pltpu.VMEM((CH,), jnp.int32),
            pltpu.VMEM((ECH_TAIL,), jnp.int32),
            pltpu.VMEM((CH,), jnp.float32),
            pltpu.VMEM((1008,), jnp.float32),
            pltpu.VMEM_SHARED((N,), jnp.float32),
            pltpu.SemaphoreType.DMA,
            pltpu.SemaphoreType.DMA,
        ],
    )
    def k(dst_hbm, out_hbm, slab, dca, dcb, dct, ones, zbuf, acc, sem_a, sem_b):
        c = lax.axis_index("c")
        s = lax.axis_index("s")
        wid = c * NS + s

        slab_cp = pltpu.async_copy(dst_hbm.at[wid], slab, sem_a)

        def zfill(j, carry):
            zbuf[pl.ds(j * 16, 16)] = jnp.zeros((16,), jnp.float32)
            return carry

        lax.fori_loop(0, 63, zfill, 0)

        def ofill(j, carry):
            ones[pl.ds(j * 16, 16)] = jnp.ones((16,), jnp.float32)
            return carry

        lax.fori_loop(0, CH // 16, ofill, 0)

        @pl.when(s < 10)
        def _():
            pltpu.sync_copy(zbuf.at[pl.ds(0, 1000)], acc.at[pl.ds(s * 1000, 1000)])

        slab_cp.wait()
        plsc.subcore_barrier()

        def cpidx(i, buf, n):
            def one(j, carry):
                buf[pl.ds(j * 16, 16)] = slab[pl.ds(i * CH + j * 16, 16)]
                return carry

            lax.fori_loop(0, n // 16, one, 0)

        def scat_start(i, buf, sem):
            cpidx(i, buf, CH)
            pltpu.async_copy(ones, acc.at[buf], sem, add=True)

        def scat_wait(buf, sem):
            pltpu.make_async_copy(dst_hbm.at[wid, pl.ds(0, CH)], buf, sem).wait()

        scat_start(0, dca, sem_a)

        def body(j, carry):
            i0 = 2 * j
            scat_start(i0 + 1, dcb, sem_b)
            scat_wait(dca, sem_a)

            @pl.when(i0 + 2 < N_CHUNKS)
            def _():
                scat_start(i0 + 2, dca, sem_a)

            scat_wait(dcb, sem_b)
            return carry

        lax.fori_loop(0, N_CHUNKS // 2, body, 0)
        dct[pl.ds(0, 16)] = slab[pl.ds(N_CHUNKS * CH, 16)]
        pltpu.sync_copy(ones.at[pl.ds(0, ECH_TAIL)], acc.at[dct], add=True)
        plsc.subcore_barrier()

        @pl.when(s == 0)
        def _():
            pltpu.sync_copy(acc, out_hbm.at[c])

    return k(dst)


def _make_sc_agg(d):
    @functools.partial(
        pl.kernel,
        mesh=_mesh(),
        out_type=jax.ShapeDtypeStruct((NC, N, d), jnp.float32),
        scratch_types=[
            pltpu.VMEM((E_PER_TILE,), jnp.int32),
            pltpu.VMEM((CH,), jnp.int32),
            pltpu.VMEM((CH,), jnp.int32),
            pltpu.VMEM((CH,), jnp.int32),
            pltpu.VMEM((CH,), jnp.int32),
            pltpu.VMEM((ECH_TAIL,), jnp.int32),
            pltpu.VMEM((ECH_TAIL,), jnp.int32),
            pltpu.VMEM((CH, d), jnp.float32),
            pltpu.VMEM((CH, d), jnp.float32),
            pltpu.VMEM((ZR, d), jnp.float32),
            pltpu.VMEM_SHARED((N, d), jnp.float32),
            pltpu.SemaphoreType.DMA,
            pltpu.SemaphoreType.DMA,
            pltpu.SemaphoreType.DMA,
            pltpu.SemaphoreType.DMA,
            pltpu.SemaphoreType.DMA,
        ],
    )
    def k(y_hbm, src_hbm, dst_hbm, out_hbm, srcv, sca, scb, dca, dcb,
          sct, dct, rows_a, rows_b, zbuf, acc, sem_a, sem_b, dsem_a, dsem_b,
          zsem):
        c = lax.axis_index("c")
        s = lax.axis_index("s")
        wid = c * NS + s

        idx_cp_s = pltpu.async_copy(src_hbm.at[wid], srcv, sem_a)

        def zrow(r, carry):
            def zcol(j, carry2):
                zbuf[r, pl.ds(j * 16, 16)] = jnp.zeros((16,), jnp.float32)
                return carry2

            return lax.fori_loop(0, d // 16, zcol, carry)

        lax.fori_loop(0, ZR, zrow, 0)

        def zstart(t, carry):
            pltpu.async_copy(zbuf, acc.at[pl.ds(s * RS + t * ZR, ZR)], zsem)
            return carry

        lax.fori_loop(0, NZ, zstart, 0)

        @pl.when(s == 0)
        def _():
            pltpu.sync_copy(zbuf.at[pl.ds(0, TAIL)], acc.at[pl.ds(NS * RS, TAIL)])

        def zdrain(t, carry):
            pltpu.make_async_copy(y_hbm.at[pl.ds(0, ZR)], zbuf, zsem).wait()
            return carry

        lax.fori_loop(0, NZ, zdrain, 0)
        idx_cp_s.wait()
        plsc.subcore_barrier()

        def cpidx(i, buf):
            def one(j, carry):
                buf[pl.ds(j * 16, 16)] = srcv[pl.ds(i * CH + j * 16, 16)]
                return carry

            lax.fori_loop(0, CH // 16, one, 0)

        def stage(i, sbuf, rows, sem, dbuf, dsem):
            cpidx(i, sbuf)
            pltpu.async_copy(y_hbm.at[sbuf], rows, sem)
            pltpu.async_copy(dst_hbm.at[wid, pl.ds(i * CH, CH)], dbuf, dsem)

        def finish(rows, sem, dbuf, dsem):
            pltpu.make_async_copy(y_hbm.at[pl.ds(0, CH)], rows, sem).wait()
            pltpu.make_async_copy(dst_hbm.at[wid, pl.ds(0, CH)], dbuf, dsem).wait()
            pltpu.sync_copy(rows, acc.at[dbuf], add=True)

        stage(0, sca, rows_a, sem_a, dca, dsem_a)

        def body(j, carry):
            i0 = 2 * j
            stage(i0 + 1, scb, rows_b, sem_b, dcb, dsem_b)
            finish(rows_a, sem_a, dca, dsem_a)

            @pl.when(i0 + 2 < N_CHUNKS)
            def _():
                stage(i0 + 2, sca, rows_a, sem_a, dca, dsem_a)

            finish(rows_b, sem_b, dcb, dsem_b)
            return carry

        lax.fori_loop(0, N_CHUNKS // 2, body, 0)

        def tcp(j, carry):
            sct[pl.ds(j * 16, 16)] = srcv[pl.ds(N_CHUNKS * CH + j * 16, 16)]
            return carry

        lax.fori_loop(0, ECH_TAIL // 16, tcp, 0)
        pltpu.sync_copy(
            dst_hbm.at[wid, pl.ds(N_CHUNKS * CH, ECH_TAIL)], dct)
        pltpu.async_copy(
            y_hbm.at[sct], rows_a.at[pl.ds(0, ECH_TAIL)], sem_a).wait()
        pltpu.sync_copy(rows_a.at[pl.ds(0, ECH_TAIL)], acc.at[dct], add=True)
        plsc.subcore_barrier()

        pltpu.sync_copy(
            acc.at[pl.ds(s * RS, RS)],
            out_hbm.at[c, pl.ds(s * RS, RS), :],
        )

        @pl.when(s == 0)
        def _():
            pltpu.sync_copy(
                acc.at[pl.ds(NS * RS, TAIL)],
                out_hbm.at[c, pl.ds(NS * RS, TAIL), :],
            )

    return k


_sc_agg128 = _make_sc_agg(D_MID)


BN = 1000
GRID = N // BN


def _elu(a):
    return jnp.where(a > 0.0, a, jnp.exp(jnp.minimum(a, 0.0)) - 1.0)


def _rows(i):
    return (i, 0)


def _fixed(i):
    return (0, 0)


def _rows3(i):
    return (0, i, 0)


def _tc1(deg, x, W1, Wl1, bl1):
    def body(deg_r, x_r, w_r, wl_r, bl_r, y_r, skip_r, dinv_r):
        dinv = lax.rsqrt(deg_r[...])
        xb = x_r[...]
        y_r[...] = dinv * jnp.dot(xb, w_r[...], preferred_element_type=jnp.float32)
        skip_r[...] = jnp.dot(xb, wl_r[...], preferred_element_type=jnp.float32) + bl_r[...]
        dinv_r[...] = dinv

    return pl.pallas_call(
        body,
        grid=(GRID,),
        in_specs=[
            pl.BlockSpec((BN, 1), _rows),
            pl.BlockSpec((BN, D_IN), _rows),
            pl.BlockSpec((D_IN, D_MID), _fixed),
            pl.BlockSpec((D_IN, D_MID), _fixed),
            pl.BlockSpec((1, D_MID), _fixed),
        ],
        out_specs=[
            pl.BlockSpec((BN, D_MID), _rows),
            pl.BlockSpec((BN, D_MID), _rows),
            pl.BlockSpec((BN, 1), _rows),
        ],
        out_shape=[
            jax.ShapeDtypeStruct((N, D_MID), jnp.float32),
            jax.ShapeDtypeStruct((N, D_MID), jnp.float32),
            jax.ShapeDtypeStruct((N, 1), jnp.float32),
        ],
    )(deg, x, W1, Wl1, bl1)


def _make_tc_mid(d_in, d_out):
    def body(u_r, y_r, dinv_r, skip_r, b_r, w_r, wl_r, bl_r, y2_r, skip2_r):
        dinv = dinv_r[...]
        u = u_r[0] + u_r[1]
        h = _elu(dinv * (u + y_r[...]) + b_r[...]) + skip_r[...]
        y2_r[...] = dinv * jnp.dot(h, w_r[...], preferred_element_type=jnp.float32)
        skip2_r[...] = jnp.dot(h, wl_r[...], preferred_element_type=jnp.float32) + bl_r[...]

    def run(u, y, dinv, skip, b, W, Wl, bl):
        return pl.pallas_call(
            body,
            grid=(GRID,),
            in_specs=[
                pl.BlockSpec((NC, BN, d_in), _rows3),
                pl.BlockSpec((BN, d_in), _rows),
                pl.BlockSpec((BN, 1), _rows),
                pl.BlockSpec((BN, d_in), _rows),
                pl.BlockSpec((1, d_in), _fixed),
                pl.BlockSpec((d_in, d_out), _fixed),
                pl.BlockSpec((d_in, d_out), _fixed),
                pl.BlockSpec((1, d_out), _fixed),
            ],
            out_specs=[
                pl.BlockSpec((BN, d_out), _rows),
                pl.BlockSpec((BN, d_out), _rows),
            ],
            out_shape=[
                jax.ShapeDtypeStruct((N, d_out), jnp.float32),
                jax.ShapeDtypeStruct((N, d_out), jnp.float32),
            ],
        )(u, y, dinv, skip, b, W, Wl, bl)

    return run


_tc_mid = _make_tc_mid(D_MID, D_MID)


def _tc_final(u, y, dinv, skip, b):
    def body(u_r, y_r, dinv_r, skip_r, b_r, o_r):
        u_ = u_r[0] + u_r[1]
        o_r[...] = dinv_r[...] * (u_ + y_r[...]) + b_r[...] + skip_r[...]

    return pl.pallas_call(
        body,
        grid=(GRID,),
        in_specs=[
            pl.BlockSpec((NC, BN, D_OUT), _rows3),
            pl.BlockSpec((BN, D_OUT), _rows),
            pl.BlockSpec((BN, 1), _rows),
            pl.BlockSpec((BN, D_OUT), _rows),
            pl.BlockSpec((1, D_OUT), _fixed),
        ],
        out_specs=pl.BlockSpec((BN, D_OUT), _rows),
        out_shape=jax.ShapeDtypeStruct((N, D_OUT), jnp.float32),
    )(u, y, dinv, skip, b)


def kernel(node_feature, adj_list, W1, b1, Wl1, bl1, W2, b2, Wl2, bl2,
           W3, b3, Wl3, bl3):
    x = node_feature
    src = adj_list[0].astype(jnp.int32)
    dst = adj_list[1].astype(jnp.int32)
    src3 = src.reshape(TILES, E_PER_TILE)
    dst3 = dst.reshape(TILES, E_PER_TILE)

    degp = _sc_degree(dst3)
    deg = (degp[0] + degp[1] + 1.0).reshape(N, 1)

    pad = D_MID - D_MID2
    W2p = jnp.pad(W2, ((0, 0), (0, pad)))
    Wl2p = jnp.pad(Wl2, ((0, 0), (0, pad)))
    W3p = jnp.pad(W3, ((0, pad), (0, 0)))
    Wl3p = jnp.pad(Wl3, ((0, pad), (0, 0)))
    b1r = b1.reshape(1, -1)
    b2r = jnp.pad(b2, (0, pad)).reshape(1, -1)
    b3r = b3.reshape(1, -1)
    bl1r = bl1.reshape(1, -1)
    bl2r = jnp.pad(bl2, (0, pad)).reshape(1, -1)
    bl3r = bl3.reshape(1, -1)

    y1, skip1, dinv = _tc1(deg, x, W1, Wl1, bl1r)
    u1 = _sc_agg128(y1, src3, dst3)
    y2, skip2 = _tc_mid(u1, y1, dinv, skip1, b1r, W2p, Wl2p, bl2r)
    u2 = _sc_agg128(y2, src3, dst3)
    y3, skip3 = _tc_mid(u2, y2, dinv, skip2, b2r, W3p, Wl3p, bl3r)
    u3 = _sc_agg128(y3, src3, dst3)
    out = _tc_final(u3, y3, dinv, skip3, b3r)
    return out
